# baseline (speedup 1.0000x reference)
import jax
import jax.numpy as jnp
from jax.experimental import pallas as pl
from jax.experimental.pallas import tpu as pltpu

N = 512
IN_DIM = 128


def _dot(a, b):
    return jax.lax.dot(a, b, preferred_element_type=jnp.float32)


def _dot_tn(a, b):
    return jax.lax.dot_general(
        a, b, (((0,), (0,)), ((), ())), preferred_element_type=jnp.float32)


def _dot_nt(a, b):
    return jax.lax.dot_general(
        a, b, (((1,), (1,)), ((), ())), preferred_element_type=jnp.float32)


def _gae_decoder_kernel(x3_hbm, adj3_hbm, Ss_hbm, W1_hbm, b1_hbm,
                        W2_hbm, b2_hbm, W3_hbm, b3_hbm, out_ref,
                        x3_v, adj3_v, S0_v, S1_v, S2_v,
                        W1_v, b1_v, W2_v, b2_v, W3_v, b3_v, sems):
    cp = pltpu.make_async_copy
    copies = [
        cp(Ss_hbm.at[2], S2_v, sems.at[0]),
        cp(adj3_hbm, adj3_v, sems.at[1]),
        cp(x3_hbm, x3_v, sems.at[2]),
        cp(W1_hbm, W1_v, sems.at[3]),
        cp(b1_hbm, b1_v, sems.at[4]),
        cp(Ss_hbm.at[1], S1_v, sems.at[5]),
        cp(W2_hbm, W2_v, sems.at[6]),
        cp(b2_hbm, b2_v, sems.at[7]),
        cp(Ss_hbm.at[0], S0_v, sems.at[8]),
        cp(W3_hbm, W3_v, sems.at[9]),
        cp(b3_hbm, b3_v, sems.at[10]),
    ]
    for c in copies:
        c.start()

    ones = jnp.ones((N, 1), dtype=jnp.float32)

    def gcn_out(A, h, b):
        deg = _dot_tn(A, ones)
        dinv = jnp.where(deg > 0, jax.lax.rsqrt(deg), 0.0)
        return jax.nn.relu(dinv * _dot_tn(A, dinv * h) + b)

    for c in copies[:5]:
        c.wait()
    S2 = S2_v[...].astype(jnp.float32)
    A3 = _dot_nt(_dot(S2, adj3_v[...].astype(jnp.float32)), S2)
    SW1 = _dot(S2, W1_v[...])
    x2_bar = gcn_out(A3, _dot(x3_v[...].astype(jnp.float32), SW1), b1_v[...])

    for c in copies[5:8]:
        c.wait()
    S1 = S1_v[...].astype(jnp.float32)
    A2 = _dot_nt(_dot(S1, A3), S1)
    SW2 = _dot(S1, W2_v[...])
    x1_bar = gcn_out(A2, _dot(x2_bar, SW2), b2_v[...])

    for c in copies[8:]:
        c.wait()
    S0 = S0_v[...].astype(jnp.float32)
    A1 = _dot_nt(_dot(S0, A2), S0)
    SW3 = _dot(S0, W3_v[...])
    out_ref[...] = gcn_out(A1, _dot(x1_bar, SW3), b3_v[...])


def kernel(x3_bar, adj3, Ss, W1, b1, W2, b2, W3, b3):
    f32 = jnp.float32
    bf16 = jnp.bfloat16
    any_spec = pl.BlockSpec(memory_space=pl.ANY)
    return pl.pallas_call(
        _gae_decoder_kernel,
        in_specs=[any_spec] * 9,
        out_specs=pl.BlockSpec(memory_space=pltpu.VMEM),
        out_shape=jax.ShapeDtypeStruct((N, IN_DIM), f32),
        scratch_shapes=[
            pltpu.VMEM((N, N), bf16),
            pltpu.VMEM((N, N), bf16),
            pltpu.VMEM((N, N), bf16),
            pltpu.VMEM((N, N), bf16),
            pltpu.VMEM((N, N), bf16),
            pltpu.VMEM((N, N), f32),
            pltpu.VMEM((1, N), f32),
            pltpu.VMEM((N, N), f32),
            pltpu.VMEM((1, N), f32),
            pltpu.VMEM((N, IN_DIM), f32),
            pltpu.VMEM((1, IN_DIM), f32),
            pltpu.SemaphoreType.DMA((11,)),
        ],
    )(x3_bar.astype(bf16), adj3.astype(bf16), Ss.astype(bf16),
      W1, b1.reshape(1, N),
      W2, b2.reshape(1, N),
      W3, b3.reshape(1, IN_DIM))


# speedup vs baseline: 1.4548x; 1.4548x over previous
import jax
import jax.numpy as jnp
from jax.experimental import pallas as pl
from jax.experimental.pallas import tpu as pltpu

N = 512
IN_DIM = 128
H = N // 2


def _dot(a, b):
    return jax.lax.dot(a, b, preferred_element_type=jnp.float32)


def _dot_tn(a, b):
    return jax.lax.dot_general(
        a, b, (((0,), (0,)), ((), ())), preferred_element_type=jnp.float32)


def _dot_nt(a, b):
    return jax.lax.dot_general(
        a, b, (((1,), (1,)), ((), ())), preferred_element_type=jnp.float32)


def _gae_decoder_kernel(x3_hbm, adj3_hbm, Ss_hbm, W1_hbm, b1_hbm,
                        W2_hbm, b2_hbm, W3_hbm, b3_hbm, out_ref,
                        x3_v, adj3_v, S0_v, S1_v, S2_v,
                        W1_v, b1_v, W2_v, b2_v, W3_v, b3_v, sems):
    cp = pltpu.make_async_copy
    c_s2 = cp(Ss_hbm.at[2], S2_v, sems.at[0])
    c_a1 = cp(adj3_hbm.at[:, 0:H], adj3_v.at[:, 0:H], sems.at[1])
    c_a2 = cp(adj3_hbm.at[:, H:N], adj3_v.at[:, H:N], sems.at[2])
    c_w1 = cp(W1_hbm, W1_v, sems.at[3])
    c_x3 = cp(x3_hbm, x3_v, sems.at[4])
    c_b1 = cp(b1_hbm, b1_v, sems.at[5])
    c_s1 = cp(Ss_hbm.at[1], S1_v, sems.at[6])
    c_w2 = cp(W2_hbm, W2_v, sems.at[7])
    c_b2 = cp(b2_hbm, b2_v, sems.at[8])
    c_s0 = cp(Ss_hbm.at[0], S0_v, sems.at[9])
    c_w3 = cp(W3_hbm, W3_v, sems.at[10])
    c_b3 = cp(b3_hbm, b3_v, sems.at[11])
    for c in (c_s2, c_a1, c_a2, c_w1, c_x3, c_b1,
              c_s1, c_w2, c_b2, c_s0, c_w3, c_b3):
        c.start()

    ones = jnp.ones((N, 1), dtype=jnp.float32)

    def gcn_out(A, h, b):
        deg = _dot_tn(A, ones)
        dinv = jnp.where(deg > 0, jax.lax.rsqrt(deg), 0.0)
        return jax.nn.relu(dinv * _dot_tn(A, dinv * h) + b)

    c_s2.wait()
    S2 = S2_v[...]
    c_a1.wait()
    T1 = _dot(S2, adj3_v[:, 0:H])
    c_a2.wait()
    T2 = _dot(S2, adj3_v[:, H:N])
    A3 = _dot_nt(T1, S2[:, 0:H]) + _dot_nt(T2, S2[:, H:N])
    c_w1.wait()
    SW1 = _dot(S2, W1_v[...])
    c_x3.wait()
    h1 = _dot(x3_v[...], SW1)
    c_b1.wait()
    x2_bar = gcn_out(A3, h1, b1_v[...])

    c_s1.wait()
    S1 = S1_v[...]
    A2 = _dot_nt(_dot(S1, A3), S1)
    c_w2.wait()
    SW2 = _dot(S1, W2_v[...])
    c_b2.wait()
    x1_bar = gcn_out(A2, _dot(x2_bar, SW2), b2_v[...])

    c_s0.wait()
    S0 = S0_v[...]
    A1 = _dot_nt(_dot(S0, A2), S0)
    c_w3.wait()
    SW3 = _dot(S0, W3_v[...])
    c_b3.wait()
    out_ref[...] = gcn_out(A1, _dot(x1_bar, SW3), b3_v[...])


def kernel(x3_bar, adj3, Ss, W1, b1, W2, b2, W3, b3):
    f32 = jnp.float32
    any_spec = pl.BlockSpec(memory_space=pl.ANY)
    return pl.pallas_call(
        _gae_decoder_kernel,
        in_specs=[any_spec] * 9,
        out_specs=pl.BlockSpec(memory_space=pltpu.VMEM),
        out_shape=jax.ShapeDtypeStruct((N, IN_DIM), f32),
        scratch_shapes=[
            pltpu.VMEM((N, N), f32),
            pltpu.VMEM((N, N), f32),
            pltpu.VMEM((N, N), f32),
            pltpu.VMEM((N, N), f32),
            pltpu.VMEM((N, N), f32),
            pltpu.VMEM((N, N), f32),
            pltpu.VMEM((1, N), f32),
            pltpu.VMEM((N, N), f32),
            pltpu.VMEM((1, N), f32),
            pltpu.VMEM((N, IN_DIM), f32),
            pltpu.VMEM((1, IN_DIM), f32),
            pltpu.SemaphoreType.DMA((12,)),
        ],
    )(x3_bar, adj3, Ss,
      W1, b1.reshape(1, N),
      W2, b2.reshape(1, N),
      W3, b3.reshape(1, IN_DIM))


# speedup vs baseline: 1.5699x; 1.0791x over previous
import jax
import jax.numpy as jnp
from jax.experimental import pallas as pl
from jax.experimental.pallas import tpu as pltpu

N = 512
IN_DIM = 128


def _dot(a, b):
    return jax.lax.dot(a, b, preferred_element_type=jnp.float32)


def _dot_tn(a, b):
    return jax.lax.dot_general(
        a, b, (((0,), (0,)), ((), ())), preferred_element_type=jnp.float32)


def _dot_nt(a, b):
    return jax.lax.dot_general(
        a, b, (((1,), (1,)), ((), ())), preferred_element_type=jnp.float32)


def _gae_decoder_kernel(x3_hbm, adj3_hbm, Ss_hbm, W1_hbm, b1_hbm,
                        W2_hbm, b2_hbm, W3_hbm, b3_hbm, out_ref,
                        x3_v, adj3_v, S0_v, S1_v, S2_v,
                        W1_v, b1_v, W2_v, b2_v, W3_v, b3_v, sems):
    cp = pltpu.make_async_copy
    copies = [
        cp(Ss_hbm.at[2], S2_v, sems.at[0]),
        cp(adj3_hbm, adj3_v, sems.at[1]),
        cp(x3_hbm, x3_v, sems.at[2]),
        cp(W1_hbm, W1_v, sems.at[3]),
        cp(b1_hbm, b1_v, sems.at[4]),
        cp(Ss_hbm.at[1], S1_v, sems.at[5]),
        cp(W2_hbm, W2_v, sems.at[6]),
        cp(b2_hbm, b2_v, sems.at[7]),
        cp(Ss_hbm.at[0], S0_v, sems.at[8]),
        cp(W3_hbm, W3_v, sems.at[9]),
        cp(b3_hbm, b3_v, sems.at[10]),
    ]
    for c in copies:
        c.start()

    ones = jnp.ones((N, 1), dtype=jnp.float32)

    def gcn_out(A, h, b):
        deg = _dot_tn(A, ones)
        dinv = jnp.where(deg > 0, jax.lax.rsqrt(deg), 0.0)
        return jax.nn.relu(dinv * _dot_tn(A, dinv * h) + b)

    for c in copies[:5]:
        c.wait()
    S2 = S2_v[...]
    A3 = _dot_nt(_dot(S2, adj3_v[...]), S2)
    SW1 = _dot(S2, W1_v[...])
    x2_bar = gcn_out(A3, _dot(x3_v[...], SW1), b1_v[...])

    for c in copies[5:8]:
        c.wait()
    S1 = S1_v[...]
    A2 = _dot_nt(_dot(S1, A3), S1)
    SW2 = _dot(S1, W2_v[...])
    x1_bar = gcn_out(A2, _dot(x2_bar, SW2), b2_v[...])

    for c in copies[8:]:
        c.wait()
    S0 = S0_v[...]
    A1 = _dot_nt(_dot(S0, A2), S0)
    SW3 = _dot(S0, W3_v[...])
    out_ref[...] = gcn_out(A1, _dot(x1_bar, SW3), b3_v[...])


def kernel(x3_bar, adj3, Ss, W1, b1, W2, b2, W3, b3):
    f32 = jnp.float32
    any_spec = pl.BlockSpec(memory_space=pl.ANY)
    return pl.pallas_call(
        _gae_decoder_kernel,
        in_specs=[any_spec] * 9,
        out_specs=pl.BlockSpec(memory_space=pltpu.VMEM),
        out_shape=jax.ShapeDtypeStruct((N, IN_DIM), f32),
        scratch_shapes=[
            pltpu.VMEM((N, N), f32),
            pltpu.VMEM((N, N), f32),
            pltpu.VMEM((N, N), f32),
            pltpu.VMEM((N, N), f32),
            pltpu.VMEM((N, N), f32),
            pltpu.VMEM((N, N), f32),
            pltpu.VMEM((N,), f32),
            pltpu.VMEM((N, N), f32),
            pltpu.VMEM((N,), f32),
            pltpu.VMEM((N, IN_DIM), f32),
            pltpu.VMEM((IN_DIM,), f32),
            pltpu.SemaphoreType.DMA((11,)),
        ],
    )(x3_bar, adj3, Ss, W1, b1, W2, b2, W3, b3)
